# baseline (speedup 1.0000x reference)
import jax
import jax.numpy as jnp
from jax.experimental import pallas as pl
from jax.experimental.pallas import tpu as pltpu


BT = 2048
K = 336


def _split(v):
    hi = v.astype(jnp.bfloat16)
    lo = (v - hi.astype(jnp.float32)).astype(jnp.bfloat16)
    return hi, lo


def _body(x_ref, emb_proto_ref, emb_flags_ref, emb_dir_ref, u_len_ref,
          u_iat_ref, w_fusion_ref, b_fusion_ref, gamma_ref, beta_ref,
          o_ref, t_ref):
    step = pl.program_id(0)

    @pl.when(step == 0)
    def _build_tables():
        wf = w_fusion_ref[...]
        tp = jnp.dot(emb_proto_ref[...], wf[0:32, :],
                     preferred_element_type=jnp.float32)
        t_ref[0:256, :] = tp.astype(jnp.bfloat16)
        tf = jnp.dot(emb_flags_ref[...], wf[32:64, :],
                     preferred_element_type=jnp.float32)
        t_ref[256:320, :] = tf.astype(jnp.bfloat16)
        td = jnp.dot(emb_dir_ref[...], wf[64:72, :],
                     preferred_element_type=jnp.float32)
        ul = jnp.dot(u_len_ref[...], wf[72:104, :],
                     preferred_element_type=jnp.float32)
        ui = jnp.dot(u_iat_ref[...], wf[104:136, :],
                     preferred_element_type=jnp.float32)
        base = td[0:1, :] + b_fusion_ref[...] + ul[1:2, :] + ui[1:2, :]
        ddiff = td[1:2, :] - td[0:1, :]
        b_hi, b_lo = _split(base)
        d_hi, d_lo = _split(ddiff)
        vl_hi, vl_lo = _split(ul[0:1, :])
        vi_hi, vi_lo = _split(ui[0:1, :])
        t_ref[320:321, :] = b_hi
        t_ref[321:322, :] = b_lo
        t_ref[322:323, :] = d_hi
        t_ref[323:324, :] = d_lo
        t_ref[324:325, :] = vl_hi
        t_ref[325:326, :] = vl_lo
        t_ref[326:327, :] = vl_hi
        t_ref[327:328, :] = vi_hi
        t_ref[328:329, :] = vi_lo
        t_ref[329:330, :] = vi_hi
        t_ref[330:336, :] = jnp.zeros((6, 256), jnp.bfloat16)

    xb = x_ref[...]
    p = jnp.clip(xb[:, 0:1].astype(jnp.int32), 0, 255)
    f = jnp.clip(xb[:, 4:5].astype(jnp.int32), 0, 63)
    d = jnp.clip(xb[:, 3:4].astype(jnp.int32), 0, 1).astype(jnp.bfloat16)

    oh_p = (p == jax.lax.broadcasted_iota(jnp.int32, (BT, 256), 1)
            ).astype(jnp.bfloat16)
    oh_f = (f == jax.lax.broadcasted_iota(jnp.int32, (BT, 64), 1)
            ).astype(jnp.bfloat16)
    l_hi, l_lo = _split(xb[:, 1:2])
    i_hi, i_lo = _split(xb[:, 2:3])
    ones = jnp.ones((BT, 1), jnp.bfloat16)
    zeros = jnp.zeros((BT, 6), jnp.bfloat16)
    oh = jnp.concatenate(
        [oh_p, oh_f, ones, ones, d, d, l_hi, l_hi, l_lo, i_hi, i_hi, i_lo,
         zeros], axis=1)

    h = jnp.dot(oh, t_ref[...], preferred_element_type=jnp.float32)

    mean = jnp.mean(h, axis=-1, keepdims=True)
    c = h - mean
    var = jnp.mean(c * c, axis=-1, keepdims=True)
    o_ref[...] = (c * jax.lax.rsqrt(var + 1e-5)) * gamma_ref[...] + beta_ref[...]


def kernel(x, emb_proto, emb_flags, emb_dir, w_len, b_len, w_iat, b_iat,
           w_fusion, b_fusion, gamma, beta):
    B, S, _ = x.shape
    n = B * S
    d_model = w_fusion.shape[1]
    xf = x.reshape(n, 5)

    emb_dir_p = jnp.pad(emb_dir, ((0, 8 - emb_dir.shape[0]), (0, 0)))
    u_len = jnp.concatenate(
        [w_len, b_len[None, :], jnp.zeros((6, 32), jnp.float32)], axis=0)
    u_iat = jnp.concatenate(
        [w_iat, b_iat[None, :], jnp.zeros((6, 32), jnp.float32)], axis=0)

    steps = n // BT
    full = lambda shape: pl.BlockSpec(shape, lambda i: (0, 0))
    out = pl.pallas_call(
        _body,
        grid=(steps,),
        in_specs=[
            pl.BlockSpec((BT, 5), lambda i: (i, 0)),
            full(emb_proto.shape),
            full(emb_flags.shape),
            full(emb_dir_p.shape),
            full(u_len.shape),
            full(u_iat.shape),
            full(w_fusion.shape),
            full((1, d_model)),
            full((1, d_model)),
            full((1, d_model)),
        ],
        out_specs=pl.BlockSpec((BT, d_model), lambda i: (i, 0)),
        out_shape=jax.ShapeDtypeStruct((n, d_model), jnp.float32),
        scratch_shapes=[
            pltpu.VMEM((K, 256), jnp.bfloat16),
        ],
    )(xf, emb_proto, emb_flags, emb_dir_p, u_len, u_iat, w_fusion,
      b_fusion[None, :], gamma[None, :], beta[None, :])
    return out.reshape(B, S, d_model)


# speedup vs baseline: 1.4487x; 1.4487x over previous
import jax
import jax.numpy as jnp
from jax.experimental import pallas as pl
from jax.experimental.pallas import tpu as pltpu


BT = 2048
K = 336


def _body(xt_ref, emb_proto_ref, emb_flags_ref, emb_dir_ref, u_len_ref,
          u_iat_ref, w_fusion_ref, b_fusion_ref, gamma_ref, beta_ref,
          o_ref, t_ref):
    step = pl.program_id(0)

    @pl.when(step == 0)
    def _build_tables():
        def center(m):
            return m - jnp.mean(m, axis=1, keepdims=True)

        wf = w_fusion_ref[...]
        tp = jnp.dot(emb_proto_ref[...], wf[0:32, :],
                     preferred_element_type=jnp.float32)
        t_ref[0:256, :] = center(tp).astype(jnp.bfloat16)
        tf = jnp.dot(emb_flags_ref[...], wf[32:64, :],
                     preferred_element_type=jnp.float32)
        t_ref[256:320, :] = center(tf).astype(jnp.bfloat16)
        td = jnp.dot(emb_dir_ref[...], wf[64:72, :],
                     preferred_element_type=jnp.float32)
        ul = jnp.dot(u_len_ref[...], wf[72:104, :],
                     preferred_element_type=jnp.float32)
        ui = jnp.dot(u_iat_ref[...], wf[104:136, :],
                     preferred_element_type=jnp.float32)
        base = td[0:1, :] + b_fusion_ref[...] + ul[1:2, :] + ui[1:2, :]
        ddiff = td[1:2, :] - td[0:1, :]
        t_ref[320:321, :] = center(base).astype(jnp.bfloat16)
        t_ref[321:322, :] = center(ddiff).astype(jnp.bfloat16)
        vl = center(ul[0:1, :]).astype(jnp.bfloat16)
        vi = center(ui[0:1, :]).astype(jnp.bfloat16)
        t_ref[322:323, :] = vl
        t_ref[323:324, :] = vl
        t_ref[324:325, :] = vi
        t_ref[325:326, :] = vi
        t_ref[326:K, :] = jnp.zeros((K - 326, 256), jnp.bfloat16)

    xr = xt_ref[...]
    pr = jnp.clip(jnp.floor(xr[0:1, :]), 0.0, 255.0).astype(jnp.int32)
    fr = jnp.clip(jnp.floor(xr[4:5, :]), 0.0, 63.0).astype(jnp.int32)
    dr = jnp.clip(jnp.floor(xr[3:4, :]), 0.0, 1.0)
    l_hi = xr[1:2, :].astype(jnp.bfloat16).astype(jnp.float32)
    l_lo = xr[1:2, :] - l_hi
    i_hi = xr[2:3, :].astype(jnp.bfloat16).astype(jnp.float32)
    i_lo = xr[2:3, :] - i_hi
    ones = jnp.ones((1, BT), jnp.float32)
    pack = jnp.concatenate([ones, dr, l_hi, l_lo, i_hi, i_lo,
                            jnp.zeros((2, BT), jnp.float32)],
                           axis=0)
    colp = pack.T
    ipack = jnp.concatenate([pr, fr, jnp.zeros((6, BT), jnp.int32)], axis=0)
    icolp = ipack.T

    oh_p = (icolp[:, 0:1] == jax.lax.broadcasted_iota(jnp.int32, (BT, 256), 1)
            ).astype(jnp.bfloat16)
    oh_f = (icolp[:, 1:2] == jax.lax.broadcasted_iota(jnp.int32, (BT, 64), 1)
            ).astype(jnp.bfloat16)
    extras = colp[:, 0:6].astype(jnp.bfloat16)
    zpad = jnp.zeros((BT, K - 326), jnp.bfloat16)
    oh = jnp.concatenate([oh_p, oh_f, extras, zpad], axis=1)

    c = jnp.dot(oh, t_ref[...], preferred_element_type=jnp.float32)

    var = jnp.mean(c * c, axis=-1, keepdims=True)
    o_ref[...] = (c * jax.lax.rsqrt(var + 1e-5)) * gamma_ref[...] + beta_ref[...]


def kernel(x, emb_proto, emb_flags, emb_dir, w_len, b_len, w_iat, b_iat,
           w_fusion, b_fusion, gamma, beta):
    B, S, _ = x.shape
    n = B * S
    d_model = w_fusion.shape[1]
    xt = jnp.pad(x.reshape(n, 5).T, ((0, 3), (0, 0)))

    emb_dir_p = jnp.pad(emb_dir, ((0, 8 - emb_dir.shape[0]), (0, 0)))
    u_len = jnp.concatenate(
        [w_len, b_len[None, :], jnp.zeros((6, 32), jnp.float32)], axis=0)
    u_iat = jnp.concatenate(
        [w_iat, b_iat[None, :], jnp.zeros((6, 32), jnp.float32)], axis=0)

    steps = n // BT
    full = lambda shape: pl.BlockSpec(shape, lambda i: (0, 0))
    out = pl.pallas_call(
        _body,
        grid=(steps,),
        in_specs=[
            pl.BlockSpec((8, BT), lambda i: (0, i)),
            full(emb_proto.shape),
            full(emb_flags.shape),
            full(emb_dir_p.shape),
            full(u_len.shape),
            full(u_iat.shape),
            full(w_fusion.shape),
            full((1, d_model)),
            full((1, d_model)),
            full((1, d_model)),
        ],
        out_specs=pl.BlockSpec((BT, d_model), lambda i: (i, 0)),
        out_shape=jax.ShapeDtypeStruct((n, d_model), jnp.float32),
        scratch_shapes=[
            pltpu.VMEM((K, 256), jnp.bfloat16),
        ],
    )(xt, emb_proto, emb_flags, emb_dir_p, u_len, u_iat, w_fusion,
      b_fusion[None, :], gamma[None, :], beta[None, :])
    return out.reshape(B, S, d_model)


# speedup vs baseline: 1.4970x; 1.0333x over previous
import jax
import jax.numpy as jnp
from jax.experimental import pallas as pl
from jax.experimental.pallas import tpu as pltpu


BT = 2048
K = 336


def _body(xt_ref, emb_proto_ref, emb_flags_ref, emb_dir_ref, u_len_ref,
          u_iat_ref, w_fusion_ref, b_fusion_ref, gamma_ref, beta_ref,
          o_ref, t_ref):
    step = pl.program_id(0)

    @pl.when(step == 0)
    def _build_tables():
        def center(m):
            return m - jnp.mean(m, axis=1, keepdims=True)

        wf = w_fusion_ref[...]
        tp = jnp.dot(emb_proto_ref[...], wf[0:32, :],
                     preferred_element_type=jnp.float32)
        t_ref[0:256, :] = center(tp).astype(jnp.bfloat16)
        tf = jnp.dot(emb_flags_ref[...], wf[32:64, :],
                     preferred_element_type=jnp.float32)
        t_ref[256:320, :] = center(tf).astype(jnp.bfloat16)
        td = jnp.dot(emb_dir_ref[...], wf[64:72, :],
                     preferred_element_type=jnp.float32)
        ul = jnp.dot(u_len_ref[...], wf[72:104, :],
                     preferred_element_type=jnp.float32)
        ui = jnp.dot(u_iat_ref[...], wf[104:136, :],
                     preferred_element_type=jnp.float32)
        base = td[0:1, :] + b_fusion_ref[...] + ul[1:2, :] + ui[1:2, :]
        ddiff = td[1:2, :] - td[0:1, :]
        t_ref[320:321, :] = center(base).astype(jnp.bfloat16)
        t_ref[321:322, :] = center(ddiff).astype(jnp.bfloat16)
        vl = center(ul[0:1, :]).astype(jnp.bfloat16)
        vi = center(ui[0:1, :]).astype(jnp.bfloat16)
        t_ref[322:323, :] = vl
        t_ref[323:324, :] = vl
        t_ref[324:325, :] = vi
        t_ref[325:326, :] = vi
        t_ref[326:K, :] = jnp.zeros((K - 326, 256), jnp.bfloat16)

    HB = BT // 2
    xr = xt_ref[...]

    def make_oh(xh):
        pr = jnp.clip(jnp.floor(xh[0:1, :]), 0.0, 255.0).astype(jnp.int32)
        fr = jnp.clip(jnp.floor(xh[4:5, :]), 0.0, 63.0).astype(jnp.int32)
        dr = jnp.clip(jnp.floor(xh[3:4, :]), 0.0, 1.0)
        l_hi = xh[1:2, :].astype(jnp.bfloat16).astype(jnp.float32)
        l_lo = xh[1:2, :] - l_hi
        i_hi = xh[2:3, :].astype(jnp.bfloat16).astype(jnp.float32)
        i_lo = xh[2:3, :] - i_hi
        ones = jnp.ones((1, HB), jnp.float32)
        pack = jnp.concatenate([ones, dr, l_hi, l_lo, i_hi, i_lo,
                                jnp.zeros((2, HB), jnp.float32)],
                               axis=0)
        colp = pack.T
        ipack = jnp.concatenate([pr, fr, jnp.zeros((6, HB), jnp.int32)],
                                axis=0)
        icolp = ipack.T
        oh_p = (icolp[:, 0:1]
                == jax.lax.broadcasted_iota(jnp.int32, (HB, 256), 1)
                ).astype(jnp.bfloat16)
        oh_f = (icolp[:, 1:2]
                == jax.lax.broadcasted_iota(jnp.int32, (HB, 64), 1)
                ).astype(jnp.bfloat16)
        extras = colp[:, 0:6].astype(jnp.bfloat16)
        zpad = jnp.zeros((HB, K - 326), jnp.bfloat16)
        return jnp.concatenate([oh_p, oh_f, extras, zpad], axis=1)

    def norm_out(c):
        var = jnp.mean(c * c, axis=-1, keepdims=True)
        return (c * jax.lax.rsqrt(var + 1e-5)) * gamma_ref[...] + beta_ref[...]

    t = t_ref[...]
    oh0 = make_oh(xr[:, 0:HB])
    c0 = jnp.dot(oh0, t, preferred_element_type=jnp.float32)
    oh1 = make_oh(xr[:, HB:BT])
    c1 = jnp.dot(oh1, t, preferred_element_type=jnp.float32)
    o_ref[0:HB, :] = norm_out(c0)
    o_ref[HB:BT, :] = norm_out(c1)


def kernel(x, emb_proto, emb_flags, emb_dir, w_len, b_len, w_iat, b_iat,
           w_fusion, b_fusion, gamma, beta):
    B, S, _ = x.shape
    n = B * S
    d_model = w_fusion.shape[1]
    xt = jnp.pad(x.reshape(n, 5).T, ((0, 3), (0, 0)))

    emb_dir_p = jnp.pad(emb_dir, ((0, 8 - emb_dir.shape[0]), (0, 0)))
    u_len = jnp.concatenate(
        [w_len, b_len[None, :], jnp.zeros((6, 32), jnp.float32)], axis=0)
    u_iat = jnp.concatenate(
        [w_iat, b_iat[None, :], jnp.zeros((6, 32), jnp.float32)], axis=0)

    steps = n // BT
    full = lambda shape: pl.BlockSpec(shape, lambda i: (0, 0))
    out = pl.pallas_call(
        _body,
        grid=(steps,),
        in_specs=[
            pl.BlockSpec((8, BT), lambda i: (0, i)),
            full(emb_proto.shape),
            full(emb_flags.shape),
            full(emb_dir_p.shape),
            full(u_len.shape),
            full(u_iat.shape),
            full(w_fusion.shape),
            full((1, d_model)),
            full((1, d_model)),
            full((1, d_model)),
        ],
        out_specs=pl.BlockSpec((BT, d_model), lambda i: (i, 0)),
        out_shape=jax.ShapeDtypeStruct((n, d_model), jnp.float32),
        scratch_shapes=[
            pltpu.VMEM((K, 256), jnp.bfloat16),
        ],
    )(xt, emb_proto, emb_flags, emb_dir_p, u_len, u_iat, w_fusion,
      b_fusion[None, :], gamma[None, :], beta[None, :])
    return out.reshape(B, S, d_model)


# speedup vs baseline: 1.5504x; 1.0357x over previous
import jax
import jax.numpy as jnp
from jax.experimental import pallas as pl
from jax.experimental.pallas import tpu as pltpu


BT = 8192
K = 336


def _body(xt_ref, emb_proto_ref, emb_flags_ref, emb_dir_ref, u_len_ref,
          u_iat_ref, w_fusion_ref, b_fusion_ref, gamma_ref, beta_ref,
          o_ref, t_ref):
    step = pl.program_id(0)

    @pl.when(step == 0)
    def _build_tables():
        def center(m):
            return m - jnp.mean(m, axis=1, keepdims=True)

        wf = w_fusion_ref[...]
        tp = jnp.dot(emb_proto_ref[...], wf[0:32, :],
                     preferred_element_type=jnp.float32)
        t_ref[0:256, :] = center(tp).astype(jnp.bfloat16)
        tf = jnp.dot(emb_flags_ref[...], wf[32:64, :],
                     preferred_element_type=jnp.float32)
        t_ref[256:320, :] = center(tf).astype(jnp.bfloat16)
        td = jnp.dot(emb_dir_ref[...], wf[64:72, :],
                     preferred_element_type=jnp.float32)
        ul = jnp.dot(u_len_ref[...], wf[72:104, :],
                     preferred_element_type=jnp.float32)
        ui = jnp.dot(u_iat_ref[...], wf[104:136, :],
                     preferred_element_type=jnp.float32)
        base = td[0:1, :] + b_fusion_ref[...] + ul[1:2, :] + ui[1:2, :]
        ddiff = td[1:2, :] - td[0:1, :]
        t_ref[320:321, :] = center(base).astype(jnp.bfloat16)
        t_ref[321:322, :] = center(ddiff).astype(jnp.bfloat16)
        vl = center(ul[0:1, :]).astype(jnp.bfloat16)
        vi = center(ui[0:1, :]).astype(jnp.bfloat16)
        t_ref[322:323, :] = vl
        t_ref[323:324, :] = vl
        t_ref[324:325, :] = vi
        t_ref[325:326, :] = vi
        t_ref[326:K, :] = jnp.zeros((K - 326, 256), jnp.bfloat16)

    HB = min(2048, BT)
    xr = xt_ref[...]

    def make_oh(xh):
        pr = jnp.clip(jnp.floor(xh[0:1, :]), 0.0, 255.0).astype(jnp.int32)
        fr = jnp.clip(jnp.floor(xh[4:5, :]), 0.0, 63.0).astype(jnp.int32)
        dr = jnp.clip(jnp.floor(xh[3:4, :]), 0.0, 1.0)
        l_hi = xh[1:2, :].astype(jnp.bfloat16).astype(jnp.float32)
        l_lo = xh[1:2, :] - l_hi
        i_hi = xh[2:3, :].astype(jnp.bfloat16).astype(jnp.float32)
        i_lo = xh[2:3, :] - i_hi
        ones = jnp.ones((1, HB), jnp.float32)
        pack = jnp.concatenate([ones, dr, l_hi, l_lo, i_hi, i_lo,
                                jnp.zeros((2, HB), jnp.float32)],
                               axis=0)
        colp = pack.T
        ipack = jnp.concatenate([pr, fr, jnp.zeros((6, HB), jnp.int32)],
                                axis=0)
        icolp = ipack.T
        oh_p = (icolp[:, 0:1]
                == jax.lax.broadcasted_iota(jnp.int32, (HB, 256), 1)
                ).astype(jnp.bfloat16)
        oh_f = (icolp[:, 1:2]
                == jax.lax.broadcasted_iota(jnp.int32, (HB, 64), 1)
                ).astype(jnp.bfloat16)
        extras = colp[:, 0:6].astype(jnp.bfloat16)
        zpad = jnp.zeros((HB, K - 326), jnp.bfloat16)
        return jnp.concatenate([oh_p, oh_f, extras, zpad], axis=1)

    def norm_out(c):
        var = jnp.mean(c * c, axis=-1, keepdims=True)
        return (c * jax.lax.rsqrt(var + 1e-5)) * gamma_ref[...] + beta_ref[...]

    t = t_ref[...]
    cs = []
    for j in range(BT // HB):
        oh = make_oh(xr[:, j * HB:(j + 1) * HB])
        cs.append(jnp.dot(oh, t, preferred_element_type=jnp.float32))
    for j, c in enumerate(cs):
        o_ref[j * HB:(j + 1) * HB, :] = norm_out(c)


def kernel(x, emb_proto, emb_flags, emb_dir, w_len, b_len, w_iat, b_iat,
           w_fusion, b_fusion, gamma, beta):
    B, S, _ = x.shape
    n = B * S
    d_model = w_fusion.shape[1]
    xt = jnp.pad(x.reshape(n, 5).T, ((0, 3), (0, 0)))

    emb_dir_p = jnp.pad(emb_dir, ((0, 8 - emb_dir.shape[0]), (0, 0)))
    u_len = jnp.concatenate(
        [w_len, b_len[None, :], jnp.zeros((6, 32), jnp.float32)], axis=0)
    u_iat = jnp.concatenate(
        [w_iat, b_iat[None, :], jnp.zeros((6, 32), jnp.float32)], axis=0)

    steps = n // BT
    full = lambda shape: pl.BlockSpec(shape, lambda i: (0, 0))
    out = pl.pallas_call(
        _body,
        grid=(steps,),
        in_specs=[
            pl.BlockSpec((8, BT), lambda i: (0, i)),
            full(emb_proto.shape),
            full(emb_flags.shape),
            full(emb_dir_p.shape),
            full(u_len.shape),
            full(u_iat.shape),
            full(w_fusion.shape),
            full((1, d_model)),
            full((1, d_model)),
            full((1, d_model)),
        ],
        out_specs=pl.BlockSpec((BT, d_model), lambda i: (i, 0)),
        out_shape=jax.ShapeDtypeStruct((n, d_model), jnp.float32),
        scratch_shapes=[
            pltpu.VMEM((K, 256), jnp.bfloat16),
        ],
    )(xt, emb_proto, emb_flags, emb_dir_p, u_len, u_iat, w_fusion,
      b_fusion[None, :], gamma[None, :], beta[None, :])
    return out.reshape(B, S, d_model)


# speedup vs baseline: 1.5543x; 1.0025x over previous
import jax
import jax.numpy as jnp
from jax.experimental import pallas as pl
from jax.experimental.pallas import tpu as pltpu


BT = 16384
K = 336


def _body(xt_ref, emb_proto_ref, emb_flags_ref, emb_dir_ref, u_len_ref,
          u_iat_ref, w_fusion_ref, b_fusion_ref, gamma_ref, beta_ref,
          o_ref, t_ref):
    step = pl.program_id(0)

    @pl.when(step == 0)
    def _build_tables():
        def center(m):
            return m - jnp.mean(m, axis=1, keepdims=True)

        wf = w_fusion_ref[...]
        tp = jnp.dot(emb_proto_ref[...], wf[0:32, :],
                     preferred_element_type=jnp.float32)
        t_ref[0:256, :] = center(tp).astype(jnp.bfloat16)
        tf = jnp.dot(emb_flags_ref[...], wf[32:64, :],
                     preferred_element_type=jnp.float32)
        t_ref[256:320, :] = center(tf).astype(jnp.bfloat16)
        td = jnp.dot(emb_dir_ref[...], wf[64:72, :],
                     preferred_element_type=jnp.float32)
        ul = jnp.dot(u_len_ref[...], wf[72:104, :],
                     preferred_element_type=jnp.float32)
        ui = jnp.dot(u_iat_ref[...], wf[104:136, :],
                     preferred_element_type=jnp.float32)
        base = td[0:1, :] + b_fusion_ref[...] + ul[1:2, :] + ui[1:2, :]
        ddiff = td[1:2, :] - td[0:1, :]
        t_ref[320:321, :] = center(base).astype(jnp.bfloat16)
        t_ref[321:322, :] = center(ddiff).astype(jnp.bfloat16)
        vl = center(ul[0:1, :]).astype(jnp.bfloat16)
        vi = center(ui[0:1, :]).astype(jnp.bfloat16)
        t_ref[322:323, :] = vl
        t_ref[323:324, :] = vl
        t_ref[324:325, :] = vi
        t_ref[325:326, :] = vi
        t_ref[326:K, :] = jnp.zeros((K - 326, 256), jnp.bfloat16)

    HB = min(2048, BT)
    xr = xt_ref[...]

    def make_oh(xh):
        pr = jnp.clip(jnp.floor(xh[0:1, :]), 0.0, 255.0).astype(jnp.int32)
        fr = jnp.clip(jnp.floor(xh[4:5, :]), 0.0, 63.0).astype(jnp.int32)
        dr = jnp.clip(jnp.floor(xh[3:4, :]), 0.0, 1.0)
        l_hi = xh[1:2, :].astype(jnp.bfloat16).astype(jnp.float32)
        l_lo = xh[1:2, :] - l_hi
        i_hi = xh[2:3, :].astype(jnp.bfloat16).astype(jnp.float32)
        i_lo = xh[2:3, :] - i_hi
        ones = jnp.ones((1, HB), jnp.float32)
        pack = jnp.concatenate([ones, dr, l_hi, l_lo, i_hi, i_lo,
                                jnp.zeros((2, HB), jnp.float32)],
                               axis=0)
        colp = pack.T
        ipack = jnp.concatenate([pr, fr, jnp.zeros((6, HB), jnp.int32)],
                                axis=0)
        icolp = ipack.T
        oh_p = (icolp[:, 0:1]
                == jax.lax.broadcasted_iota(jnp.int32, (HB, 256), 1)
                ).astype(jnp.bfloat16)
        oh_f = (icolp[:, 1:2]
                == jax.lax.broadcasted_iota(jnp.int32, (HB, 64), 1)
                ).astype(jnp.bfloat16)
        extras = colp[:, 0:6].astype(jnp.bfloat16)
        zpad = jnp.zeros((HB, K - 326), jnp.bfloat16)
        return jnp.concatenate([oh_p, oh_f, extras, zpad], axis=1)

    def norm_out(c):
        var = jnp.mean(c * c, axis=-1, keepdims=True)
        return (c * jax.lax.rsqrt(var + 1e-5)) * gamma_ref[...] + beta_ref[...]

    t = t_ref[...]
    nsub = BT // HB
    for jp in range(0, nsub, 2):
        pair = range(jp, min(jp + 2, nsub))
        cs = []
        for j in pair:
            oh = make_oh(xr[:, j * HB:(j + 1) * HB])
            cs.append(jnp.dot(oh, t, preferred_element_type=jnp.float32))
        for j, c in zip(pair, cs):
            o_ref[j * HB:(j + 1) * HB, :] = norm_out(c)


def kernel(x, emb_proto, emb_flags, emb_dir, w_len, b_len, w_iat, b_iat,
           w_fusion, b_fusion, gamma, beta):
    B, S, _ = x.shape
    n = B * S
    d_model = w_fusion.shape[1]
    xt = jnp.pad(x.reshape(n, 5).T, ((0, 3), (0, 0)))

    emb_dir_p = jnp.pad(emb_dir, ((0, 8 - emb_dir.shape[0]), (0, 0)))
    u_len = jnp.concatenate(
        [w_len, b_len[None, :], jnp.zeros((6, 32), jnp.float32)], axis=0)
    u_iat = jnp.concatenate(
        [w_iat, b_iat[None, :], jnp.zeros((6, 32), jnp.float32)], axis=0)

    steps = n // BT
    full = lambda shape: pl.BlockSpec(shape, lambda i: (0, 0))
    out = pl.pallas_call(
        _body,
        grid=(steps,),
        in_specs=[
            pl.BlockSpec((8, BT), lambda i: (0, i)),
            full(emb_proto.shape),
            full(emb_flags.shape),
            full(emb_dir_p.shape),
            full(u_len.shape),
            full(u_iat.shape),
            full(w_fusion.shape),
            full((1, d_model)),
            full((1, d_model)),
            full((1, d_model)),
        ],
        out_specs=pl.BlockSpec((BT, d_model), lambda i: (i, 0)),
        out_shape=jax.ShapeDtypeStruct((n, d_model), jnp.float32),
        scratch_shapes=[
            pltpu.VMEM((K, 256), jnp.bfloat16),
        ],
    )(xt, emb_proto, emb_flags, emb_dir_p, u_len, u_iat, w_fusion,
      b_fusion[None, :], gamma[None, :], beta[None, :])
    return out.reshape(B, S, d_model)


# speedup vs baseline: 1.7916x; 1.1527x over previous
import jax
import jax.numpy as jnp
from jax.experimental import pallas as pl
from jax.experimental.pallas import tpu as pltpu


BT = 16384
K = 336


def _body(xt_ref, emb_proto_ref, emb_flags_ref, emb_dir_ref, u_len_ref,
          u_iat_ref, w_fusion_ref, b_fusion_ref, gamma_ref, beta_ref,
          iota_p_ref, iota_f_ref, o_ref, t_ref):
    step = pl.program_id(0)

    @pl.when(step == 0)
    def _build_tables():
        def center(m):
            return m - jnp.mean(m, axis=1, keepdims=True)

        wf = w_fusion_ref[...]
        tp = jnp.dot(emb_proto_ref[...], wf[0:32, :],
                     preferred_element_type=jnp.float32)
        t_ref[0:256, :] = center(tp).astype(jnp.bfloat16)
        tf = jnp.dot(emb_flags_ref[...], wf[32:64, :],
                     preferred_element_type=jnp.float32)
        t_ref[256:320, :] = center(tf).astype(jnp.bfloat16)
        td = jnp.dot(emb_dir_ref[...], wf[64:72, :],
                     preferred_element_type=jnp.float32)
        ul = jnp.dot(u_len_ref[...], wf[72:104, :],
                     preferred_element_type=jnp.float32)
        ui = jnp.dot(u_iat_ref[...], wf[104:136, :],
                     preferred_element_type=jnp.float32)
        base = td[0:1, :] + b_fusion_ref[...] + ul[1:2, :] + ui[1:2, :]
        ddiff = td[1:2, :] - td[0:1, :]
        t_ref[320:321, :] = center(base).astype(jnp.bfloat16)
        t_ref[321:322, :] = center(ddiff).astype(jnp.bfloat16)
        vl = center(ul[0:1, :]).astype(jnp.bfloat16)
        vi = center(ui[0:1, :]).astype(jnp.bfloat16)
        t_ref[322:323, :] = vl
        t_ref[323:324, :] = vl
        t_ref[324:325, :] = vi
        t_ref[325:326, :] = vi
        t_ref[326:K, :] = jnp.zeros((K - 326, 256), jnp.bfloat16)

    HB = min(4096, BT)
    xr = xt_ref[...]

    def make_oh(xh, iota_p, iota_f):
        pr = jnp.clip(jnp.floor(xh[0:1, :]), 0.0, 255.0).astype(jnp.bfloat16)
        fr = jnp.clip(jnp.floor(xh[4:5, :]), 0.0, 63.0).astype(jnp.bfloat16)
        dr = jnp.clip(jnp.floor(xh[3:4, :]), 0.0, 1.0).astype(jnp.bfloat16)
        l_hi = xh[1:2, :].astype(jnp.bfloat16)
        l_lo = (xh[1:2, :] - l_hi.astype(jnp.float32)).astype(jnp.bfloat16)
        i_hi = xh[2:3, :].astype(jnp.bfloat16)
        i_lo = (xh[2:3, :] - i_hi.astype(jnp.float32)).astype(jnp.bfloat16)
        ones = jnp.ones((1, HB), jnp.bfloat16)
        pack = jnp.concatenate([pr, fr, ones, dr, l_hi, l_lo, i_hi, i_lo],
                               axis=0)
        colp = pack.T
        oh_p = (colp[:, 0:1] == iota_p).astype(jnp.bfloat16)
        oh_f = (colp[:, 1:2] == iota_f).astype(jnp.bfloat16)
        zpad = jnp.zeros((HB, K - 326), jnp.bfloat16)
        return jnp.concatenate([oh_p, oh_f, colp[:, 2:8], zpad],
                               axis=1)

    def norm_out(c):
        var = jnp.mean(c * c, axis=-1, keepdims=True)
        return (c * jax.lax.rsqrt(var + 1e-5)) * gamma_ref[...] + beta_ref[...]

    t = t_ref[...]
    iota_p = iota_p_ref[...]
    iota_f = iota_f_ref[...]
    nsub = BT // HB
    for jp in range(0, nsub, 2):
        pair = range(jp, min(jp + 2, nsub))
        cs = []
        for j in pair:
            oh = make_oh(xr[:, j * HB:(j + 1) * HB], iota_p, iota_f)
            cs.append(jnp.dot(oh, t, preferred_element_type=jnp.float32))
        for j, c in zip(pair, cs):
            o_ref[j * HB:(j + 1) * HB, :] = norm_out(c)


def kernel(x, emb_proto, emb_flags, emb_dir, w_len, b_len, w_iat, b_iat,
           w_fusion, b_fusion, gamma, beta):
    B, S, _ = x.shape
    n = B * S
    d_model = w_fusion.shape[1]
    xt = jnp.pad(x.reshape(n, 5).T, ((0, 3), (0, 0)))

    emb_dir_p = jnp.pad(emb_dir, ((0, 8 - emb_dir.shape[0]), (0, 0)))
    u_len = jnp.concatenate(
        [w_len, b_len[None, :], jnp.zeros((6, 32), jnp.float32)], axis=0)
    u_iat = jnp.concatenate(
        [w_iat, b_iat[None, :], jnp.zeros((6, 32), jnp.float32)], axis=0)

    steps = n // BT
    full = lambda shape: pl.BlockSpec(shape, lambda i: (0, 0))
    out = pl.pallas_call(
        _body,
        grid=(steps,),
        in_specs=[
            pl.BlockSpec((8, BT), lambda i: (0, i)),
            full(emb_proto.shape),
            full(emb_flags.shape),
            full(emb_dir_p.shape),
            full(u_len.shape),
            full(u_iat.shape),
            full(w_fusion.shape),
            full((1, d_model)),
            full((1, d_model)),
            full((1, d_model)),
            full((1, 256)),
            full((1, 64)),
        ],
        out_specs=pl.BlockSpec((BT, d_model), lambda i: (i, 0)),
        out_shape=jax.ShapeDtypeStruct((n, d_model), jnp.float32),
        scratch_shapes=[
            pltpu.VMEM((K, 256), jnp.bfloat16),
        ],
    )(xt, emb_proto, emb_flags, emb_dir_p, u_len, u_iat, w_fusion,
      b_fusion[None, :], gamma[None, :], beta[None, :],
      jnp.arange(256, dtype=jnp.bfloat16)[None, :],
      jnp.arange(64, dtype=jnp.bfloat16)[None, :])
    return out.reshape(B, S, d_model)


# speedup vs baseline: 1.7930x; 1.0008x over previous
import jax
import jax.numpy as jnp
from jax.experimental import pallas as pl
from jax.experimental.pallas import tpu as pltpu


BT = 8192
K = 336


def _body(xt_ref, emb_proto_ref, emb_flags_ref, emb_dir_ref, u_len_ref,
          u_iat_ref, w_fusion_ref, b_fusion_ref, gamma_ref, beta_ref,
          iota_p_ref, iota_f_ref, o_ref, t_ref):
    step = pl.program_id(0)

    @pl.when(step == 0)
    def _build_tables():
        def center(m):
            return m - jnp.mean(m, axis=1, keepdims=True)

        wf = w_fusion_ref[...]
        tp = jnp.dot(emb_proto_ref[...], wf[0:32, :],
                     preferred_element_type=jnp.float32)
        t_ref[0:256, :] = center(tp).astype(jnp.bfloat16)
        tf = jnp.dot(emb_flags_ref[...], wf[32:64, :],
                     preferred_element_type=jnp.float32)
        t_ref[256:320, :] = center(tf).astype(jnp.bfloat16)
        td = jnp.dot(emb_dir_ref[...], wf[64:72, :],
                     preferred_element_type=jnp.float32)
        ul = jnp.dot(u_len_ref[...], wf[72:104, :],
                     preferred_element_type=jnp.float32)
        ui = jnp.dot(u_iat_ref[...], wf[104:136, :],
                     preferred_element_type=jnp.float32)
        base = td[0:1, :] + b_fusion_ref[...] + ul[1:2, :] + ui[1:2, :]
        ddiff = td[1:2, :] - td[0:1, :]
        t_ref[320:321, :] = center(base).astype(jnp.bfloat16)
        t_ref[321:322, :] = center(ddiff).astype(jnp.bfloat16)
        vl = center(ul[0:1, :]).astype(jnp.bfloat16)
        vi = center(ui[0:1, :]).astype(jnp.bfloat16)
        t_ref[322:323, :] = vl
        t_ref[323:324, :] = vl
        t_ref[324:325, :] = vi
        t_ref[325:326, :] = vi
        t_ref[326:K, :] = jnp.zeros((K - 326, 256), jnp.bfloat16)

    HB = min(2048, BT)
    xr = xt_ref[...]

    def make_oh(xh, iota_p, iota_f):
        pr = jnp.clip(jnp.floor(xh[0:1, :]), 0.0, 255.0).astype(jnp.bfloat16)
        fr = jnp.clip(jnp.floor(xh[4:5, :]), 0.0, 63.0).astype(jnp.bfloat16)
        dr = jnp.clip(jnp.floor(xh[3:4, :]), 0.0, 1.0).astype(jnp.bfloat16)
        l_hi = xh[1:2, :].astype(jnp.bfloat16)
        l_lo = (xh[1:2, :] - l_hi.astype(jnp.float32)).astype(jnp.bfloat16)
        i_hi = xh[2:3, :].astype(jnp.bfloat16)
        i_lo = (xh[2:3, :] - i_hi.astype(jnp.float32)).astype(jnp.bfloat16)
        ones = jnp.ones((1, HB), jnp.bfloat16)
        pack = jnp.concatenate([pr, fr, ones, dr, l_hi, l_lo, i_hi, i_lo],
                               axis=0)
        colp = pack.T
        oh_p = (colp[:, 0:1] == iota_p).astype(jnp.bfloat16)
        oh_f = (colp[:, 1:2] == iota_f).astype(jnp.bfloat16)
        zpad = jnp.zeros((HB, K - 326), jnp.bfloat16)
        return jnp.concatenate([oh_p, oh_f, colp[:, 2:8], zpad],
                               axis=1)

    def norm_out(c):
        var = jnp.mean(c * c, axis=-1, keepdims=True)
        return c * (jax.lax.rsqrt(var + 1e-5) * gamma_ref[...]) + beta_ref[...]

    t = t_ref[...]
    iota_p = iota_p_ref[...]
    iota_f = iota_f_ref[...]
    nsub = BT // HB
    for jp in range(0, nsub, 2):
        pair = range(jp, min(jp + 2, nsub))
        cs = []
        for j in pair:
            oh = make_oh(xr[:, j * HB:(j + 1) * HB], iota_p, iota_f)
            cs.append(jnp.dot(oh, t, preferred_element_type=jnp.float32))
        for j, c in zip(pair, cs):
            o_ref[j * HB:(j + 1) * HB, :] = norm_out(c)


def kernel(x, emb_proto, emb_flags, emb_dir, w_len, b_len, w_iat, b_iat,
           w_fusion, b_fusion, gamma, beta):
    B, S, _ = x.shape
    n = B * S
    d_model = w_fusion.shape[1]
    xt = jnp.pad(x.reshape(n, 5).T, ((0, 3), (0, 0)))

    emb_dir_p = jnp.pad(emb_dir, ((0, 8 - emb_dir.shape[0]), (0, 0)))
    u_len = jnp.concatenate(
        [w_len, b_len[None, :], jnp.zeros((6, 32), jnp.float32)], axis=0)
    u_iat = jnp.concatenate(
        [w_iat, b_iat[None, :], jnp.zeros((6, 32), jnp.float32)], axis=0)

    steps = n // BT
    full = lambda shape: pl.BlockSpec(shape, lambda i: (0, 0))
    out = pl.pallas_call(
        _body,
        grid=(steps,),
        in_specs=[
            pl.BlockSpec((8, BT), lambda i: (0, i)),
            full(emb_proto.shape),
            full(emb_flags.shape),
            full(emb_dir_p.shape),
            full(u_len.shape),
            full(u_iat.shape),
            full(w_fusion.shape),
            full((1, d_model)),
            full((1, d_model)),
            full((1, d_model)),
            full((1, 256)),
            full((1, 64)),
        ],
        out_specs=pl.BlockSpec((BT, d_model), lambda i: (i, 0)),
        out_shape=jax.ShapeDtypeStruct((n, d_model), jnp.float32),
        scratch_shapes=[
            pltpu.VMEM((K, 256), jnp.bfloat16),
        ],
    )(xt, emb_proto, emb_flags, emb_dir_p, u_len, u_iat, w_fusion,
      b_fusion[None, :], gamma[None, :], beta[None, :],
      jnp.arange(256, dtype=jnp.bfloat16)[None, :],
      jnp.arange(64, dtype=jnp.bfloat16)[None, :])
    return out.reshape(B, S, d_model)


# speedup vs baseline: 1.8054x; 1.0069x over previous
import jax
import jax.numpy as jnp
from jax.experimental import pallas as pl
from jax.experimental.pallas import tpu as pltpu


BT = 16384
K = 336


def _body(xt_ref, emb_proto_ref, emb_flags_ref, emb_dir_ref, u_len_ref,
          u_iat_ref, w_fusion_ref, b_fusion_ref, gamma_ref, beta_ref,
          iota_p_ref, iota_f_ref, o_ref, t_ref):
    step = pl.program_id(0)

    @pl.when(step == 0)
    def _build_tables():
        def center(m):
            return m - jnp.mean(m, axis=1, keepdims=True)

        wf = w_fusion_ref[...]
        tp = jnp.dot(emb_proto_ref[...], wf[0:32, :],
                     preferred_element_type=jnp.float32)
        t_ref[0:256, :] = center(tp).astype(jnp.bfloat16)
        tf = jnp.dot(emb_flags_ref[...], wf[32:64, :],
                     preferred_element_type=jnp.float32)
        t_ref[256:320, :] = center(tf).astype(jnp.bfloat16)
        td = jnp.dot(emb_dir_ref[...], wf[64:72, :],
                     preferred_element_type=jnp.float32)
        ul = jnp.dot(u_len_ref[...], wf[72:104, :],
                     preferred_element_type=jnp.float32)
        ui = jnp.dot(u_iat_ref[...], wf[104:136, :],
                     preferred_element_type=jnp.float32)
        base = td[0:1, :] + b_fusion_ref[...] + ul[1:2, :] + ui[1:2, :]
        ddiff = td[1:2, :] - td[0:1, :]
        t_ref[320:321, :] = center(base).astype(jnp.bfloat16)
        t_ref[321:322, :] = center(ddiff).astype(jnp.bfloat16)
        vl = center(ul[0:1, :]).astype(jnp.bfloat16)
        vi = center(ui[0:1, :]).astype(jnp.bfloat16)
        t_ref[322:323, :] = vl
        t_ref[323:324, :] = vl
        t_ref[324:325, :] = vi
        t_ref[325:326, :] = vi
        t_ref[326:K, :] = jnp.zeros((K - 326, 256), jnp.bfloat16)

    HB = min(2048, BT)
    xr = xt_ref[...]

    def make_oh(xh, iota_p, iota_f):
        pr = jnp.clip(jnp.floor(xh[0:1, :]), 0.0, 255.0).astype(jnp.bfloat16)
        fr = jnp.clip(jnp.floor(xh[4:5, :]), 0.0, 63.0).astype(jnp.bfloat16)
        dr = jnp.clip(jnp.floor(xh[3:4, :]), 0.0, 1.0).astype(jnp.bfloat16)
        l_hi = xh[1:2, :].astype(jnp.bfloat16)
        l_lo = (xh[1:2, :] - l_hi.astype(jnp.float32)).astype(jnp.bfloat16)
        i_hi = xh[2:3, :].astype(jnp.bfloat16)
        i_lo = (xh[2:3, :] - i_hi.astype(jnp.float32)).astype(jnp.bfloat16)
        ones = jnp.ones((1, HB), jnp.bfloat16)
        pack = jnp.concatenate([pr, fr, ones, dr, l_hi, l_lo, i_hi, i_lo],
                               axis=0)
        colp = pack.T
        oh_p = (colp[:, 0:1] == iota_p).astype(jnp.bfloat16)
        oh_f = (colp[:, 1:2] == iota_f).astype(jnp.bfloat16)
        zpad = jnp.zeros((HB, K - 326), jnp.bfloat16)
        return jnp.concatenate([oh_p, oh_f, colp[:, 2:8], zpad],
                               axis=1)

    def norm_out(c):
        var = jnp.mean(c * c, axis=-1, keepdims=True)
        return c * (jax.lax.rsqrt(var + 1e-5) * gamma_ref[...]) + beta_ref[...]

    t = t_ref[...]
    iota_p = iota_p_ref[...]
    iota_f = iota_f_ref[...]
    nsub = BT // HB
    for jp in range(0, nsub, 2):
        pair = range(jp, min(jp + 2, nsub))
        cs = []
        for j in pair:
            oh = make_oh(xr[:, j * HB:(j + 1) * HB], iota_p, iota_f)
            cs.append(jnp.dot(oh, t, preferred_element_type=jnp.float32))
        for j, c in zip(pair, cs):
            o_ref[j * HB:(j + 1) * HB, :] = norm_out(c)


def kernel(x, emb_proto, emb_flags, emb_dir, w_len, b_len, w_iat, b_iat,
           w_fusion, b_fusion, gamma, beta):
    B, S, _ = x.shape
    n = B * S
    d_model = w_fusion.shape[1]
    xt = jnp.pad(x.reshape(n, 5).T, ((0, 3), (0, 0)))

    emb_dir_p = jnp.pad(emb_dir, ((0, 8 - emb_dir.shape[0]), (0, 0)))
    u_len = jnp.concatenate(
        [w_len, b_len[None, :], jnp.zeros((6, 32), jnp.float32)], axis=0)
    u_iat = jnp.concatenate(
        [w_iat, b_iat[None, :], jnp.zeros((6, 32), jnp.float32)], axis=0)

    steps = n // BT
    full = lambda shape: pl.BlockSpec(shape, lambda i: (0, 0))
    out = pl.pallas_call(
        _body,
        grid=(steps,),
        in_specs=[
            pl.BlockSpec((8, BT), lambda i: (0, i)),
            full(emb_proto.shape),
            full(emb_flags.shape),
            full(emb_dir_p.shape),
            full(u_len.shape),
            full(u_iat.shape),
            full(w_fusion.shape),
            full((1, d_model)),
            full((1, d_model)),
            full((1, d_model)),
            full((1, 256)),
            full((1, 64)),
        ],
        out_specs=pl.BlockSpec((BT, d_model), lambda i: (i, 0)),
        out_shape=jax.ShapeDtypeStruct((n, d_model), jnp.float32),
        scratch_shapes=[
            pltpu.VMEM((K, 256), jnp.bfloat16),
        ],
    )(xt, emb_proto, emb_flags, emb_dir_p, u_len, u_iat, w_fusion,
      b_fusion[None, :], gamma[None, :], beta[None, :],
      jnp.arange(256, dtype=jnp.bfloat16)[None, :],
      jnp.arange(64, dtype=jnp.bfloat16)[None, :])
    return out.reshape(B, S, d_model)
